# baseline (speedup 1.0000x reference)
import jax
import jax.numpy as jnp
from jax.experimental import pallas as pl
from jax.experimental.pallas import tpu as pltpu

N_TOK = 4096
D_MODEL = 4096
N_EXP = 64
EP = 128
CAP = 64
T_BLK = 256
GRID = N_TOK // T_BLK


def _gate_block(x_ref, wt_ref, cw_ref, mask_ref, stats_ref, cnt_ref, gsum_ref):
    i = pl.program_id(0)

    @pl.when(i == 0)
    def _init():
        cnt_ref[...] = jnp.zeros_like(cnt_ref)
        gsum_ref[...] = jnp.zeros_like(gsum_ref)

    x = x_ref[...]
    wt = wt_ref[...]
    logits = jnp.dot(x, wt, preferred_element_type=jnp.float32)
    lane = jax.lax.broadcasted_iota(jnp.int32, (T_BLK, EP), 1)
    logits = jnp.where(lane < N_EXP, logits, jnp.float32(-1e30))
    m = jnp.max(logits, axis=1, keepdims=True)
    ex = jnp.exp(logits - m)
    gates = ex / jnp.sum(ex, axis=1, keepdims=True)
    gmax = jnp.max(gates, axis=1, keepdims=True)
    eidx = jnp.min(jnp.where(gates == gmax, lane, EP), axis=1, keepdims=True)
    onehot = (lane == eidx).astype(jnp.float32)

    r = jax.lax.broadcasted_iota(jnp.int32, (T_BLK, T_BLK), 0)
    c = jax.lax.broadcasted_iota(jnp.int32, (T_BLK, T_BLK), 1)
    tri = (r >= c).astype(jnp.float32)
    cum = jnp.dot(tri, onehot, preferred_element_type=jnp.float32)

    prev = cnt_ref[...]
    pos = jnp.sum((cum - 1.0 + prev) * onehot, axis=1, keepdims=True)
    pos = pos.astype(jnp.int32)
    keep = pos < CAP
    flat = jnp.where(keep, eidx * CAP + pos, -1)

    e3 = jax.lax.broadcasted_iota(jnp.int32, (T_BLK, N_EXP, CAP), 1)
    c3 = jax.lax.broadcasted_iota(jnp.int32, (T_BLK, N_EXP, CAP), 2)
    j3 = e3 * CAP + c3
    flat3 = flat.reshape(T_BLK, 1, 1)
    gmax3 = gmax.reshape(T_BLK, 1, 1)
    hit = j3 == flat3
    cw_ref[...] = jnp.where(hit, gmax3, 0.0)
    mask_ref[...] = hit

    cnt_ref[...] = prev + cum[T_BLK - 1 : T_BLK, :]
    gsum_ref[...] = gsum_ref[...] + jnp.sum(gates, axis=0, keepdims=True)

    @pl.when(i == GRID - 1)
    def _fin():
        cnts = cnt_ref[...]
        gs = gsum_ref[...]
        laux = jnp.sum(cnts * gs) * jnp.float32(N_EXP / (N_TOK * N_TOK))
        row = jax.lax.broadcasted_iota(jnp.int32, (8, EP), 0)
        stats_ref[...] = jnp.where(
            row == 0,
            jnp.broadcast_to(cnts, (8, EP)),
            jnp.where(row == 1, jnp.broadcast_to(gs, (8, EP)), laux),
        )


def _run_gate(x, wt):
    return pl.pallas_call(
        _gate_block,
        grid=(GRID,),
        in_specs=[
            pl.BlockSpec((T_BLK, D_MODEL), lambda i: (i, 0)),
            pl.BlockSpec((D_MODEL, EP), lambda i: (0, 0)),
        ],
        out_specs=[
            pl.BlockSpec((T_BLK, N_EXP, CAP), lambda i: (i, 0, 0)),
            pl.BlockSpec((T_BLK, N_EXP, CAP), lambda i: (i, 0, 0)),
            pl.BlockSpec((8, EP), lambda i: (0, 0)),
        ],
        out_shape=[
            jax.ShapeDtypeStruct((N_TOK, N_EXP, CAP), jnp.float32),
            jax.ShapeDtypeStruct((N_TOK, N_EXP, CAP), jnp.bool_),
            jax.ShapeDtypeStruct((8, EP), jnp.float32),
        ],
        scratch_shapes=[
            pltpu.VMEM((1, EP), jnp.float32),
            pltpu.VMEM((1, EP), jnp.float32),
        ],
        compiler_params=pltpu.CompilerParams(
            dimension_semantics=("arbitrary",),
        ),
    )(x, wt)


@jax.jit
def kernel(x, W):
    wt = jnp.zeros((D_MODEL, EP), jnp.float32).at[:, :N_EXP].set(W.T)
    combine_weights, dispatch_mask, stats = _run_gate(x, wt)
    l_aux = stats[2, 0]
    exp_counts = stats[0, :N_EXP].astype(jnp.int32)
    return (l_aux, combine_weights, dispatch_mask, exp_counts)


# speedup vs baseline: 3.6709x; 3.6709x over previous
import jax
import jax.numpy as jnp
from jax.experimental import pallas as pl
from jax.experimental.pallas import tpu as pltpu

N_TOK = 4096
D_MODEL = 4096
N_EXP = 64
EP = 128
CAP = 64
T_BLK = 256
GRID = N_TOK // T_BLK


def _gate_block(x_ref, w_ref, cw_ref, mask_ref, stats_ref, cnt_ref, gsum_ref):
    i = pl.program_id(0)

    @pl.when(i == 0)
    def _init():
        cnt_ref[...] = jnp.zeros_like(cnt_ref)
        gsum_ref[...] = jnp.zeros_like(gsum_ref)

    x = x_ref[...]
    w = w_ref[...]
    logits = jax.lax.dot_general(
        w, x, (((1,), (1,)), ((), ())), preferred_element_type=jnp.float32
    )
    sub = jax.lax.broadcasted_iota(jnp.int32, (EP, T_BLK), 0)
    logits = jnp.where(sub < N_EXP, logits, jnp.float32(-1e30))
    m = jnp.max(logits, axis=0, keepdims=True)
    ex = jnp.exp(logits - m)
    gates = ex / jnp.sum(ex, axis=0, keepdims=True)
    gmax = jnp.max(gates, axis=0, keepdims=True)
    eidx = jnp.min(jnp.where(gates == gmax, sub, EP), axis=0, keepdims=True)
    onehot = (sub == eidx).astype(jnp.float32)

    r = jax.lax.broadcasted_iota(jnp.int32, (T_BLK, T_BLK), 0)
    c = jax.lax.broadcasted_iota(jnp.int32, (T_BLK, T_BLK), 1)
    tri = (r <= c).astype(jnp.float32)
    cum = jnp.dot(onehot, tri, preferred_element_type=jnp.float32)

    prev = cnt_ref[...]
    pos = jnp.sum((cum - 1.0 + prev) * onehot, axis=0, keepdims=True)
    pos = pos.astype(jnp.int32)
    keep = pos < CAP
    flat = jnp.where(keep, eidx * CAP + pos, -1)

    e3 = jax.lax.broadcasted_iota(jnp.int32, (N_EXP, CAP, T_BLK), 0)
    c3 = jax.lax.broadcasted_iota(jnp.int32, (N_EXP, CAP, T_BLK), 1)
    j3 = e3 * CAP + c3
    flat3 = flat.reshape(1, 1, T_BLK)
    hit = j3 == flat3
    cw_ref[...] = jnp.where(hit, gmax.reshape(1, 1, T_BLK), 0.0)
    mask_ref[...] = hit.astype(jnp.int8)

    cnt_ref[...] = prev + cum[:, T_BLK - 1 : T_BLK]
    gsum_ref[...] = gsum_ref[...] + jnp.sum(gates, axis=1, keepdims=True)

    @pl.when(i == GRID - 1)
    def _fin():
        cnts = cnt_ref[...]
        gs = gsum_ref[...]
        laux = jnp.sum(cnts * gs) * jnp.float32(N_EXP / (N_TOK * N_TOK))
        lane = jax.lax.broadcasted_iota(jnp.int32, (EP, 8), 1)
        stats_ref[...] = jnp.where(
            lane == 0,
            jnp.broadcast_to(cnts, (EP, 8)),
            jnp.where(lane == 1, jnp.broadcast_to(gs, (EP, 8)), laux),
        )


def _run_gate(x, w_pad):
    return pl.pallas_call(
        _gate_block,
        grid=(GRID,),
        in_specs=[
            pl.BlockSpec((T_BLK, D_MODEL), lambda i: (i, 0)),
            pl.BlockSpec((EP, D_MODEL), lambda i: (0, 0)),
        ],
        out_specs=[
            pl.BlockSpec((N_EXP, CAP, T_BLK), lambda i: (0, 0, i)),
            pl.BlockSpec((N_EXP, CAP, T_BLK), lambda i: (0, 0, i)),
            pl.BlockSpec((EP, 8), lambda i: (0, 0)),
        ],
        out_shape=[
            jax.ShapeDtypeStruct((N_EXP, CAP, N_TOK), jnp.float32),
            jax.ShapeDtypeStruct((N_EXP, CAP, N_TOK), jnp.int8),
            jax.ShapeDtypeStruct((EP, 8), jnp.float32),
        ],
        scratch_shapes=[
            pltpu.VMEM((EP, 1), jnp.float32),
            pltpu.VMEM((EP, 1), jnp.float32),
        ],
        compiler_params=pltpu.CompilerParams(
            dimension_semantics=("arbitrary",),
        ),
    )(x, w_pad)


def _kernel_impl(x, W):
    w_pad = jnp.zeros((EP, D_MODEL), jnp.float32).at[:N_EXP].set(W)
    cw_t, mask_t, stats = _run_gate(x, w_pad)
    l_aux = stats[0, 2]
    exp_counts = stats[:N_EXP, 0].astype(jnp.int32)
    combine_weights = jnp.transpose(cw_t, (2, 0, 1))
    dispatch_mask = jnp.transpose(mask_t, (2, 0, 1)).astype(jnp.bool_)
    return (l_aux, combine_weights, dispatch_mask, exp_counts)


_probe_done = []


def kernel(x, W):
    if not _probe_done:
        _probe_done.append(1)
        try:
            txt = jax.jit(_kernel_impl).lower(x, W).compile().as_text()
            print("=== CANDIDATE HLO (layout lines) ===")
            for line in txt.splitlines():
                if ("ENTRY" in line or "sparse" in line.lower() or "copy" in line
                        or "transpose" in line or "fusion" in line):
                    print(line.strip()[:240])
        except Exception as e:
            print("probe failed:", e)
    return jax.jit(_kernel_impl)(x, W)


# speedup vs baseline: 3.7028x; 1.0087x over previous
import jax
import jax.numpy as jnp
from jax.experimental import pallas as pl
from jax.experimental.pallas import tpu as pltpu

N_TOK = 4096
D_MODEL = 4096
N_EXP = 64
EP = 128
CAP = 64
T_BLK = 512
GRID = N_TOK // T_BLK


def _gate_block(x_ref, w_ref, cw_ref, mask_ref, stats_ref, cnt_ref, gsum_ref):
    i = pl.program_id(0)

    @pl.when(i == 0)
    def _init():
        cnt_ref[...] = jnp.zeros_like(cnt_ref)
        gsum_ref[...] = jnp.zeros_like(gsum_ref)

    x = x_ref[...]
    w = w_ref[...]
    logits = jax.lax.dot_general(
        w, x, (((1,), (1,)), ((), ())), preferred_element_type=jnp.float32
    )
    sub = jax.lax.broadcasted_iota(jnp.int32, (EP, T_BLK), 0)
    logits = jnp.where(sub < N_EXP, logits, jnp.float32(-1e30))
    m = jnp.max(logits, axis=0, keepdims=True)
    ex = jnp.exp(logits - m)
    gates = ex / jnp.sum(ex, axis=0, keepdims=True)
    gmax = jnp.max(gates, axis=0, keepdims=True)
    eidx = jnp.min(jnp.where(gates == gmax, sub, EP), axis=0, keepdims=True)
    onehot = (sub == eidx).astype(jnp.float32)

    r = jax.lax.broadcasted_iota(jnp.int32, (T_BLK, T_BLK), 0)
    c = jax.lax.broadcasted_iota(jnp.int32, (T_BLK, T_BLK), 1)
    tri = (r <= c).astype(jnp.float32)
    cum = jnp.dot(onehot, tri, preferred_element_type=jnp.float32)

    prev = cnt_ref[...]
    pos = jnp.sum((cum - 1.0 + prev) * onehot, axis=0, keepdims=True)
    pos = pos.astype(jnp.int32)
    keep = pos < CAP
    flat = jnp.where(keep, eidx * CAP + pos, -1)

    e3 = jax.lax.broadcasted_iota(jnp.int32, (N_EXP, CAP, T_BLK), 0)
    c3 = jax.lax.broadcasted_iota(jnp.int32, (N_EXP, CAP, T_BLK), 1)
    j3 = e3 * CAP + c3
    flat3 = flat.reshape(1, 1, T_BLK)
    hit = j3 == flat3
    cw_ref[...] = jnp.where(hit, gmax.reshape(1, 1, T_BLK), 0.0)
    mask_ref[...] = hit.astype(jnp.int8)

    cnt_ref[...] = prev + cum[:, T_BLK - 1 : T_BLK]
    gsum_ref[...] = gsum_ref[...] + jnp.sum(gates, axis=1, keepdims=True)

    @pl.when(i == GRID - 1)
    def _fin():
        cnts = cnt_ref[...]
        gs = gsum_ref[...]
        laux = jnp.sum(cnts * gs) * jnp.float32(N_EXP / (N_TOK * N_TOK))
        lane = jax.lax.broadcasted_iota(jnp.int32, (EP, 8), 1)
        stats_ref[...] = jnp.where(
            lane == 0,
            jnp.broadcast_to(cnts, (EP, 8)),
            jnp.where(lane == 1, jnp.broadcast_to(gs, (EP, 8)), laux),
        )


def _run_gate(x, w_pad):
    return pl.pallas_call(
        _gate_block,
        grid=(GRID,),
        in_specs=[
            pl.BlockSpec((T_BLK, D_MODEL), lambda i: (i, 0)),
            pl.BlockSpec((EP, D_MODEL), lambda i: (0, 0)),
        ],
        out_specs=[
            pl.BlockSpec((N_EXP, CAP, T_BLK), lambda i: (0, 0, i)),
            pl.BlockSpec((N_EXP, CAP, T_BLK), lambda i: (0, 0, i)),
            pl.BlockSpec((EP, 8), lambda i: (0, 0)),
        ],
        out_shape=[
            jax.ShapeDtypeStruct((N_EXP, CAP, N_TOK), jnp.float32),
            jax.ShapeDtypeStruct((N_EXP, CAP, N_TOK), jnp.int8),
            jax.ShapeDtypeStruct((EP, 8), jnp.float32),
        ],
        scratch_shapes=[
            pltpu.VMEM((EP, 1), jnp.float32),
            pltpu.VMEM((EP, 1), jnp.float32),
        ],
        compiler_params=pltpu.CompilerParams(
            dimension_semantics=("arbitrary",),
        ),
    )(x, w_pad)


def _kernel_impl(x, W):
    w_pad = jnp.zeros((EP, D_MODEL), jnp.float32).at[:N_EXP].set(W)
    cw_t, mask_t, stats = _run_gate(x, w_pad)
    l_aux = stats[0, 2]
    exp_counts = stats[:N_EXP, 0].astype(jnp.int32)
    combine_weights = jnp.transpose(cw_t, (2, 0, 1))
    dispatch_mask = jnp.transpose(mask_t, (2, 0, 1)).astype(jnp.bool_)
    return (l_aux, combine_weights, dispatch_mask, exp_counts)


_probe_done = []


def kernel(x, W):
    if not _probe_done:
        _probe_done.append(1)
        try:
            txt = jax.jit(_kernel_impl).lower(x, W).compile().as_text()
            print("=== CANDIDATE HLO (layout lines) ===")
            for line in txt.splitlines():
                if ("ENTRY" in line or "sparse" in line.lower() or "copy" in line
                        or "transpose" in line or "fusion" in line):
                    print(line.strip()[:240])
        except Exception as e:
            print("probe failed:", e)
    return jax.jit(_kernel_impl)(x, W)
